# baseline (speedup 1.0000x reference)
import functools

import jax
import jax.numpy as jnp
from jax import lax
from jax.experimental import pallas as pl
from jax.experimental.pallas import tpu as pltpu
from jax.experimental.pallas import tpu_sc as plsc

NC = 2
NS = 16
NW = NC * NS
LANES = 16
CH = 832
CH3 = CH * 3
GG = 104
NCHUNK = 16
PW = CH * NCHUNK


def _mesh():
    return plsc.VectorSubcoreMesh(
        core_axis_name="c", subcore_axis_name="s", num_cores=NC,
        num_subcores=NS)


def _params():
    return pltpu.CompilerParams(
        use_tc_tiling_on_sc=False, needs_layout_passes=False)


def _make_main(NP, M):
    n_tail = M - NW * PW
    assert 0 <= n_tail <= CH and n_tail % LANES == 0
    RWB = NP // NS
    assert RWB % 8 == 0
    bsub = [CH] * (RWB // CH)
    if RWB % CH:
        bsub.append(RWB % CH)

    @functools.partial(
        pl.kernel,
        out_type=(jax.ShapeDtypeStruct((M, LANES), jnp.float32),
                  jax.ShapeDtypeStruct((2 * NC * NP, LANES), jnp.float32)),
        mesh=_mesh(),
        scratch_types=[
            pltpu.VMEM((2, 3, CH), jnp.int32),
            pltpu.VMEM((2, 3, CH), jnp.float32),
            pltpu.VMEM((2, 3, CH, LANES), jnp.float32),
            pltpu.VMEM((2, CH, LANES), jnp.float32),
            pltpu.SemaphoreType.DMA,
            pltpu.SemaphoreType.DMA,
            pltpu.SemaphoreType.DMA,
            pltpu.SemaphoreType.DMA,
        ],
        compiler_params=_params(),
    )
    def main(vt, i0, i1, i2, s0, s1, s2, out, tbl, idxv, sgnv, gbuf, obuf,
             gsem0, gsem1, osem0, osem1):
        gsem = (gsem0, gsem1)
        osem = (osem0, osem1)
        irefs = (i0, i1, i2)
        srefs = (s0, s1, s2)
        cid = lax.axis_index("c")
        sid = lax.axis_index("s")
        wid = cid * NS + sid
        wbase = wid * PW
        cb = cid * (2 * NP)
        iota = lax.iota(jnp.int32, LANES)
        sel_neg = iota * 0 + (cb + NP)
        sel_pos = iota * 0 + cb

        rb0 = sid * RWB
        off = 0
        for sz in bsub:
            pltpu.sync_copy(vt.at[pl.ds(rb0 + off, sz)],
                            gbuf.at[0, 0, pl.ds(0, sz)])

            def pa(i, carry):
                r = i * 4
                for u in range(4):
                    gbuf[0, 1, r + u] = 0.5 - 0.5 * gbuf[0, 0, r + u]
                return carry

            lax.fori_loop(0, sz // 4, pa, 0)
            pltpu.sync_copy(gbuf.at[0, 1, pl.ds(0, sz)],
                            tbl.at[pl.ds(cb + rb0 + off, sz)])

            def pb(i, carry):
                r = i * 4
                for u in range(4):
                    gbuf[0, 1, r + u] = 0.5 + 0.5 * gbuf[0, 0, r + u]
                return carry

            lax.fori_loop(0, sz // 4, pb, 0)
            pltpu.sync_copy(gbuf.at[0, 1, pl.ds(0, sz)],
                            tbl.at[pl.ds(cb + NP + rb0 + off, sz)])
            off += sz
        plsc.subcore_barrier()

        def load_fire(ci, p):
            base = wbase + ci * CH
            for k in range(3):
                pltpu.sync_copy(irefs[k].at[pl.ds(base, CH)], idxv.at[p, k])
                pltpu.sync_copy(srefs[k].at[pl.ds(base, CH)], sgnv.at[p, k])

            def abody(g, carry):
                o = g * 64
                for k in range(3):
                    for u in range(4):
                        oo = o + u * LANES
                        ii = idxv[p, k, pl.ds(oo, LANES)]
                        ss = sgnv[p, k, pl.ds(oo, LANES)]
                        idxv[p, k, pl.ds(oo, LANES)] = ii + jnp.where(
                            ss < 0.0, sel_neg, sel_pos)
                return carry

            lax.fori_loop(0, CH // 64, abody, 0)
            for k in range(3):
                for j in range(CH // GG):
                    pltpu.async_copy(
                        tbl.at[idxv.at[p, k, pl.ds(j * GG, GG)]],
                        gbuf.at[p, k, pl.ds(j * GG, GG)],
                        gsem[p])

        def wait_gather(p):
            for k in range(3):
                pltpu.make_async_copy(
                    tbl.at[pl.ds(0, CH)], gbuf.at[p, k], gsem[p]).wait()

        def compute(p):
            def cbody(i, carry):
                c = i * 4
                for u in range(4):
                    obuf[p, c + u] = jnp.minimum(
                        jnp.minimum(gbuf[p, 0, c + u], gbuf[p, 1, c + u]),
                        gbuf[p, 2, c + u])
                return carry

            lax.fori_loop(0, CH // 4, cbody, 0)

        def flush_out(ci, p):
            pltpu.async_copy(
                obuf.at[p], out.at[pl.ds(wbase + ci * CH, CH)], osem[p])

        def wait_out(p):
            pltpu.make_async_copy(
                obuf.at[p], out.at[pl.ds(0, CH)], osem[p]).wait()

        def step(ci, p, do_wait_out, next_ci):
            wait_gather(p)
            if do_wait_out:
                wait_out(p)
            compute(p)
            flush_out(ci, p)
            if next_ci is not None:
                load_fire(next_ci, p)

        load_fire(0, 0)
        load_fire(1, 1)
        step(0, 0, False, 2)
        step(1, 1, False, 3)

        def pair(t, carry):
            ca = 2 * t
            step(ca, 0, True, ca + 2)
            step(ca + 1, 1, True, ca + 3)
            return carry

        lax.fori_loop(1, NCHUNK // 2 - 1, pair, 0)
        step(NCHUNK - 2, 0, True, None)
        step(NCHUNK - 1, 1, True, None)
        wait_out(0)
        wait_out(1)

        if n_tail:
            @pl.when(wid == 0)
            def _():
                base = NW * PW
                for k in range(3):
                    pltpu.sync_copy(irefs[k].at[pl.ds(base, n_tail)],
                                    idxv.at[0, k, pl.ds(0, n_tail)])
                    pltpu.sync_copy(srefs[k].at[pl.ds(base, n_tail)],
                                    sgnv.at[0, k, pl.ds(0, n_tail)])

                def abody(g, carry):
                    o = g * LANES
                    for k in range(3):
                        ii = idxv[0, k, pl.ds(o, LANES)]
                        ss = sgnv[0, k, pl.ds(o, LANES)]
                        idxv[0, k, pl.ds(o, LANES)] = ii + jnp.where(
                            ss < 0.0, sel_neg, sel_pos)
                    return carry

                lax.fori_loop(0, n_tail // LANES, abody, 0)
                for k in range(3):
                    pltpu.async_copy(
                        tbl.at[idxv.at[0, k, pl.ds(0, n_tail)]],
                        gbuf.at[0, k, pl.ds(0, n_tail)], gsem0)
                for k in range(3):
                    pltpu.make_async_copy(
                        tbl.at[pl.ds(0, n_tail)],
                        gbuf.at[0, k, pl.ds(0, n_tail)], gsem0).wait()

                def cbody(i, carry):
                    obuf[0, i] = jnp.minimum(
                        jnp.minimum(gbuf[0, 0, i], gbuf[0, 1, i]),
                        gbuf[0, 2, i])
                    return carry

                lax.fori_loop(0, n_tail, cbody, 0)
                pltpu.async_copy(
                    obuf.at[0, pl.ds(0, n_tail)],
                    out.at[pl.ds(base, n_tail)], osem0)
                pltpu.make_async_copy(
                    obuf.at[0, pl.ds(0, n_tail)],
                    out.at[pl.ds(base, n_tail)], osem0).wait()

    return main


def kernel(v, input_idx, input_sign):
    B, N = v.shape
    M, K = input_idx.shape
    assert B == LANES and K == 3

    NP = (N + NS * 8 - 1) // (NS * 8) * (NS * 8)

    vt = jnp.zeros((NP, LANES), jnp.float32).at[:N].set(v.T)
    outT, _ = _make_main(NP, M)(
        vt,
        input_idx[:, 0], input_idx[:, 1], input_idx[:, 2],
        input_sign[:, 0], input_sign[:, 1], input_sign[:, 2])
    return outT.T
